# baseline (speedup 1.0000x reference)
import jax
import jax.numpy as jnp
from jax.experimental import pallas as pl
from jax.experimental.pallas import tpu as pltpu

N_EDGES = 320000
N_GRAPHS = 64
RATIO = 0.5
EPS = 1e-12

_BLK = 3200

_ROWS = 4096
_LANES = 128
_M = _ROWS * _LANES
_IN_ROWS = 2560
_PAD_TO = _IN_ROWS * _LANES
_IMAX = 2147483647
_CH = 32
_CHP = 256


def _mlp_body(r_ref, c_ref, w1a_ref, w1b_ref, b1_ref, w2_ref, b2_ref, att_ref):
    h = jnp.dot(r_ref[...], w1a_ref[...], preferred_element_type=jnp.float32)
    h = h + jnp.dot(c_ref[...], w1b_ref[...], preferred_element_type=jnp.float32)
    h = jax.nn.relu(h + b1_ref[...])
    att = jnp.sum(h * w2_ref[...], axis=1) + b2_ref[0, 0]
    att_ref[...] = att.reshape(1, 1, -1)


def _mlp_att(R, C, W1, b1, W2, b2):
    W1a = W1[:128]
    W1b = W1[128:]
    b1r = b1.reshape(1, -1)
    w2r = W2.reshape(1, -1)
    b2r = b2.reshape(1, 1)
    grid = N_EDGES // _BLK
    return pl.pallas_call(
        _mlp_body,
        grid=(grid,),
        in_specs=[
            pl.BlockSpec((_BLK, 128), lambda i: (i, 0)),
            pl.BlockSpec((_BLK, 128), lambda i: (i, 0)),
            pl.BlockSpec((128, 512), lambda i: (0, 0)),
            pl.BlockSpec((128, 512), lambda i: (0, 0)),
            pl.BlockSpec((1, 512), lambda i: (0, 0)),
            pl.BlockSpec((1, 512), lambda i: (0, 0)),
            pl.BlockSpec((1, 1), lambda i: (0, 0), memory_space=pltpu.SMEM),
        ],
        out_specs=pl.BlockSpec((1, 1, _BLK), lambda i: (i, 0, 0)),
        out_shape=jax.ShapeDtypeStruct((grid, 1, _BLK), jnp.float32),
    )(R, C, W1a, W1b, b1r, w2r, b2r).reshape(N_EDGES)


def _exch(K, V, kp, vp, right, desc):
    sgtp = (K > kp) | ((K == kp) & (V > vp))
    take = sgtp ^ right ^ desc
    return jnp.where(take, kp, K), jnp.where(take, vp, V)


def _sort_body(att_ref, idx_ref, mn_ref, mx_ref, perm_ref, k_ref):
    v_ref = perm_ref
    mn = mn_ref[0, 0]
    mx = mx_ref[0, 0]
    laneP = jax.lax.broadcasted_iota(jnp.int32, (1, _LANES), 1)
    lrowP = jax.lax.broadcasted_iota(jnp.int32, (_CHP, 1), 0)
    lrow = jax.lax.broadcasted_iota(jnp.int32, (_CH, 1), 0)

    def pro(q, c):
        base = pl.multiple_of(q * _CHP, _CHP)
        a = att_ref[pl.ds(base, _CHP), :]
        g = idx_ref[pl.ds(base, _CHP), :].astype(jnp.float32)
        norm = (a - mn) / (mx - mn + EPS) + g * (-1.0)
        u = jax.lax.bitcast_convert_type(norm, jnp.uint32)
        s = jnp.where(u < jnp.uint32(0x80000000), u ^ jnp.uint32(0x80000000), ~u)
        ki = jax.lax.bitcast_convert_type((~s) ^ jnp.uint32(0x80000000), jnp.int32)
        eid = (base + lrowP) * 128 + laneP
        valid = eid < N_EDGES
        k_ref[pl.ds(base, _CHP), :] = jnp.where(valid, ki, jnp.int32(_IMAX))
        v_ref[pl.ds(base, _CHP), :] = jnp.where(valid, eid, jnp.int32(_IMAX))
        return c

    jax.lax.fori_loop(0, _IN_ROWS // _CHP, pro, 0)

    padc = jnp.full((_CHP, _LANES), _IMAX, jnp.int32)

    def padf(q, c):
        base = pl.multiple_of(_IN_ROWS + q * _CHP, _CHP)
        k_ref[pl.ds(base, _CHP), :] = padc
        v_ref[pl.ds(base, _CHP), :] = padc
        return c

    jax.lax.fori_loop(0, (_ROWS - _IN_ROWS) // _CHP, padf, 0)

    lane = jax.lax.broadcasted_iota(jnp.int32, (1, _LANES), 1)

    def roll_exch(K, V, axis, r, right, desc):
        kp = jnp.where(right, jnp.roll(K, r, axis=axis), jnp.roll(K, -r, axis=axis))
        vp = jnp.where(right, jnp.roll(V, r, axis=axis), jnp.roll(V, -r, axis=axis))
        return _exch(K, V, kp, vp, right, desc)

    def phase_a(q, c):
        base = pl.multiple_of(q * _CH, _CH)
        K = k_ref[pl.ds(base, _CH), :]
        V = v_ref[pl.ds(base, _CH), :]
        for kk in range(1, 8):
            if kk < 7:
                desc = ((lane >> kk) & 1) != 0
            else:
                desc = (lrow & 1) != 0
            for j in range(min(kk - 1, 6), -1, -1):
                s2 = 1 << j
                right = (lane & s2) != 0
                K, V = roll_exch(K, V, 1, s2, right, desc)
        k_ref[pl.ds(base, _CH), :] = K
        v_ref[pl.ds(base, _CH), :] = V
        return c

    jax.lax.fori_loop(0, _ROWS // _CH, phase_a, 0)

    for kk in range(8, 20):
        for j in range(kk - 1, 9, -1):
            r = 1 << (j - 7)
            ch2 = min(r, 64)
            tpb = r // ch2

            def bigrow(q, c, r=r, ch2=ch2, tpb=tpb, kk=kk, j=j):
                bp = q // tpb
                t = q % tpb
                base = pl.multiple_of(bp * (2 * r) + t * ch2, 8)
                klo = k_ref[pl.ds(base, ch2), :]
                vlo = v_ref[pl.ds(base, ch2), :]
                khi = k_ref[pl.ds(base + r, ch2), :]
                vhi = v_ref[pl.ds(base + r, ch2), :]
                asc = ((bp >> (kk - j - 1)) & 1) == 0
                gt01 = (klo > khi) | ((klo == khi) & (vlo > vhi))
                lt01 = (khi > klo) | ((khi == klo) & (vhi > vlo))
                swap = (gt01 & asc) | (lt01 & jnp.logical_not(asc))
                k_ref[pl.ds(base, ch2), :] = jnp.where(swap, khi, klo)
                v_ref[pl.ds(base, ch2), :] = jnp.where(swap, vhi, vlo)
                k_ref[pl.ds(base + r, ch2), :] = jnp.where(swap, klo, khi)
                v_ref[pl.ds(base + r, ch2), :] = jnp.where(swap, vlo, vhi)
                return c

            jax.lax.fori_loop(0, (_ROWS // (2 * r)) * tpb, bigrow, 0)

        def tail(q, c, kk=kk):
            base = pl.multiple_of(q * _CH, _CH)
            K = k_ref[pl.ds(base, _CH), :]
            V = v_ref[pl.ds(base, _CH), :]
            kb = kk - 7
            if (1 << kb) < _CH:
                desc = (((base + lrow) >> kb) & 1) != 0
            else:
                desc = (((base >> kb) & 1) != 0)
            for j in range(min(kk - 1, 9), -1, -1):
                if j >= 7:
                    r = 1 << (j - 7)
                    right = (lrow & r) != 0
                    K, V = roll_exch(K, V, 0, r, right, desc)
                else:
                    s2 = 1 << j
                    right = (lane & s2) != 0
                    K, V = roll_exch(K, V, 1, s2, right, desc)
            k_ref[pl.ds(base, _CH), :] = K
            v_ref[pl.ds(base, _CH), :] = V
            return c

        jax.lax.fori_loop(0, _ROWS // _CH, tail, 0)


def _sort_perm(att_p, idx_p, mn, mx):
    return pl.pallas_call(
        _sort_body,
        in_specs=[
            pl.BlockSpec((_IN_ROWS, _LANES), lambda: (0, 0)),
            pl.BlockSpec((_IN_ROWS, _LANES), lambda: (0, 0)),
            pl.BlockSpec((1, 1), lambda: (0, 0), memory_space=pltpu.SMEM),
            pl.BlockSpec((1, 1), lambda: (0, 0), memory_space=pltpu.SMEM),
        ],
        out_specs=pl.BlockSpec((_ROWS, _LANES), lambda: (0, 0)),
        out_shape=jax.ShapeDtypeStruct((_ROWS, _LANES), jnp.int32),
        scratch_shapes=[
            pltpu.VMEM((_ROWS, _LANES), jnp.int32),
        ],
    )(att_p, idx_p, mn, mx)


def kernel(emb, edge_index, node_batch, W1, b1, W2, b2):
    row = edge_index[0]
    col = edge_index[1]
    R = jnp.take(emb, row, axis=0)
    C = jnp.take(emb, col, axis=0)
    att = _mlp_att(R, C, W1, b1, W2, b2)
    index = jnp.take(node_batch, row)

    mn = att.min().reshape(1, 1)
    mx = att.max().reshape(1, 1)
    att_p = jnp.pad(att, (0, _PAD_TO - N_EDGES)).reshape(_IN_ROWS, _LANES)
    idx_p = jnp.pad(index, (0, _PAD_TO - N_EDGES)).reshape(_IN_ROWS, _LANES)
    perm = _sort_perm(att_p, idx_p, mn, mx).reshape(_M)[:N_EDGES]

    deg = jnp.bincount(index, length=N_GRAPHS)
    k = jnp.ceil(RATIO * deg.astype(jnp.float32)).astype(jnp.int32)
    cum = jnp.cumsum(deg)
    start = jnp.concatenate([jnp.zeros((1,), dtype=deg.dtype), cum])
    pos = jnp.arange(N_EDGES)
    gpos = jnp.searchsorted(cum, pos, side='right')
    mask = (pos - jnp.take(start, gpos)) < jnp.take(k, gpos)
    sorted_att = jnp.take(att, perm)
    signed = jnp.where(mask, sorted_att, -sorted_att)

    ck = jnp.cumsum(k)
    k_total = ck[-1]
    cke = ck - k
    rest = deg.astype(jnp.int32) - k
    cr = jnp.cumsum(rest)
    cre = cr - rest
    gi = jnp.searchsorted(ck, pos, side='right')
    src_top = jnp.take(start, gi) + (pos - jnp.take(cke, gi))
    j2 = pos - k_total
    gj = jnp.searchsorted(cr, j2, side='right')
    src_bot = jnp.take(start, gj) + jnp.take(k, gj) + (j2 - jnp.take(cre, gj))
    order = jnp.where(pos < k_total, src_top, src_bot)
    return jnp.take(signed, order)


# speedup vs baseline: 1.0022x; 1.0022x over previous
import jax
import jax.numpy as jnp
from jax.experimental import pallas as pl
from jax.experimental.pallas import tpu as pltpu

N_EDGES = 320000
N_GRAPHS = 64
RATIO = 0.5
EPS = 1e-12

_BLK = 3200

_ROWS = 4096
_LANES = 128
_M = _ROWS * _LANES
_IN_ROWS = 2560
_PAD_TO = _IN_ROWS * _LANES
_IMAX = 2147483647
_CH = 32
_CHP = 256


def _mlp_body(r_ref, c_ref, w1a_ref, w1b_ref, b1_ref, w2_ref, b2_ref, att_ref):
    h = jnp.dot(r_ref[...], w1a_ref[...], preferred_element_type=jnp.float32)
    h = h + jnp.dot(c_ref[...], w1b_ref[...], preferred_element_type=jnp.float32)
    h = jax.nn.relu(h + b1_ref[...])
    att = jnp.sum(h * w2_ref[...], axis=1) + b2_ref[0, 0]
    att_ref[...] = att.reshape(1, 1, -1)


def _mlp_att(R, C, W1, b1, W2, b2):
    W1a = W1[:128]
    W1b = W1[128:]
    b1r = b1.reshape(1, -1)
    w2r = W2.reshape(1, -1)
    b2r = b2.reshape(1, 1)
    grid = N_EDGES // _BLK
    return pl.pallas_call(
        _mlp_body,
        grid=(grid,),
        in_specs=[
            pl.BlockSpec((_BLK, 128), lambda i: (i, 0)),
            pl.BlockSpec((_BLK, 128), lambda i: (i, 0)),
            pl.BlockSpec((128, 512), lambda i: (0, 0)),
            pl.BlockSpec((128, 512), lambda i: (0, 0)),
            pl.BlockSpec((1, 512), lambda i: (0, 0)),
            pl.BlockSpec((1, 512), lambda i: (0, 0)),
            pl.BlockSpec((1, 1), lambda i: (0, 0), memory_space=pltpu.SMEM),
        ],
        out_specs=pl.BlockSpec((1, 1, _BLK), lambda i: (i, 0, 0)),
        out_shape=jax.ShapeDtypeStruct((grid, 1, _BLK), jnp.float32),
    )(R, C, W1a, W1b, b1r, w2r, b2r).reshape(N_EDGES)


def _exch(K, V, kp, vp, right, desc):
    sgtp = (K > kp) | ((K == kp) & (V > vp))
    take = sgtp ^ right ^ desc
    return jnp.where(take, kp, K), jnp.where(take, vp, V)


def _sort_body(att_ref, idx_ref, mn_ref, mx_ref, perm_ref, k_ref):
    v_ref = perm_ref
    mn = mn_ref[0, 0]
    mx = mx_ref[0, 0]
    laneP = jax.lax.broadcasted_iota(jnp.int32, (1, _LANES), 1)
    lrowP = jax.lax.broadcasted_iota(jnp.int32, (_CHP, 1), 0)
    lrow = jax.lax.broadcasted_iota(jnp.int32, (_CH, 1), 0)

    def pro(q, c):
        base = pl.multiple_of(q * _CHP, _CHP)
        a = att_ref[pl.ds(base, _CHP), :]
        g = idx_ref[pl.ds(base, _CHP), :].astype(jnp.float32)
        norm = (a - mn) / (mx - mn + EPS) + g * (-1.0)
        u = jax.lax.bitcast_convert_type(norm, jnp.uint32)
        s = jnp.where(u < jnp.uint32(0x80000000), u ^ jnp.uint32(0x80000000), ~u)
        ki = jax.lax.bitcast_convert_type((~s) ^ jnp.uint32(0x80000000), jnp.int32)
        eid = (base + lrowP) * 128 + laneP
        valid = eid < N_EDGES
        k_ref[pl.ds(base, _CHP), :] = jnp.where(valid, ki, jnp.int32(_IMAX))
        v_ref[pl.ds(base, _CHP), :] = jnp.where(valid, eid, jnp.int32(_IMAX))
        return c

    jax.lax.fori_loop(0, _IN_ROWS // _CHP, pro, 0)

    padc = jnp.full((_CHP, _LANES), _IMAX, jnp.int32)

    def padf(q, c):
        base = pl.multiple_of(_IN_ROWS + q * _CHP, _CHP)
        k_ref[pl.ds(base, _CHP), :] = padc
        v_ref[pl.ds(base, _CHP), :] = padc
        return c

    jax.lax.fori_loop(0, (_ROWS - _IN_ROWS) // _CHP, padf, 0)

    lane = jax.lax.broadcasted_iota(jnp.int32, (1, _LANES), 1)

    def roll_exch(K, V, axis, r, right, desc):
        kp = jnp.where(right, jnp.roll(K, r, axis=axis), jnp.roll(K, -r, axis=axis))
        vp = jnp.where(right, jnp.roll(V, r, axis=axis), jnp.roll(V, -r, axis=axis))
        return _exch(K, V, kp, vp, right, desc)

    def phase_a(q, c):
        base = pl.multiple_of(q * _CH, _CH)
        K = k_ref[pl.ds(base, _CH), :]
        V = v_ref[pl.ds(base, _CH), :]
        for kk in range(1, 8):
            if kk < 7:
                desc = ((lane >> kk) & 1) != 0
            else:
                desc = (lrow & 1) != 0
            for j in range(min(kk - 1, 6), -1, -1):
                s2 = 1 << j
                right = (lane & s2) != 0
                K, V = roll_exch(K, V, 1, s2, right, desc)
        k_ref[pl.ds(base, _CH), :] = K
        v_ref[pl.ds(base, _CH), :] = V
        return c

    jax.lax.fori_loop(0, _ROWS // _CH, phase_a, 0)

    for kk in range(8, 20):
        for j in range(kk - 1, 11, -1):
            r = 1 << (j - 7)
            ch2 = min(r, 64)
            tpb = r // ch2

            def bigrow(q, c, r=r, ch2=ch2, tpb=tpb, kk=kk, j=j):
                bp = q // tpb
                t = q % tpb
                base = pl.multiple_of(bp * (2 * r) + t * ch2, 8)
                klo = k_ref[pl.ds(base, ch2), :]
                vlo = v_ref[pl.ds(base, ch2), :]
                khi = k_ref[pl.ds(base + r, ch2), :]
                vhi = v_ref[pl.ds(base + r, ch2), :]
                asc = ((bp >> (kk - j - 1)) & 1) == 0
                gt01 = (klo > khi) | ((klo == khi) & (vlo > vhi))
                lt01 = (khi > klo) | ((khi == klo) & (vhi > vlo))
                swap = (gt01 & asc) | (lt01 & jnp.logical_not(asc))
                k_ref[pl.ds(base, ch2), :] = jnp.where(swap, khi, klo)
                v_ref[pl.ds(base, ch2), :] = jnp.where(swap, vhi, vlo)
                k_ref[pl.ds(base + r, ch2), :] = jnp.where(swap, klo, khi)
                v_ref[pl.ds(base + r, ch2), :] = jnp.where(swap, vlo, vhi)
                return c

            jax.lax.fori_loop(0, (_ROWS // (2 * r)) * tpb, bigrow, 0)

        def tail(q, c, kk=kk):
            base = pl.multiple_of(q * _CH, _CH)
            K = k_ref[pl.ds(base, _CH), :]
            V = v_ref[pl.ds(base, _CH), :]
            kb = kk - 7
            if (1 << kb) < _CH:
                desc = (((base + lrow) >> kb) & 1) != 0
            else:
                desc = (((base >> kb) & 1) != 0)
            for j in range(min(kk - 1, 11), -1, -1):
                if j >= 7:
                    r = 1 << (j - 7)
                    right = (lrow & r) != 0
                    K, V = roll_exch(K, V, 0, r, right, desc)
                else:
                    s2 = 1 << j
                    right = (lane & s2) != 0
                    K, V = roll_exch(K, V, 1, s2, right, desc)
            k_ref[pl.ds(base, _CH), :] = K
            v_ref[pl.ds(base, _CH), :] = V
            return c

        jax.lax.fori_loop(0, _ROWS // _CH, tail, 0)


def _sort_perm(att_p, idx_p, mn, mx):
    return pl.pallas_call(
        _sort_body,
        in_specs=[
            pl.BlockSpec((_IN_ROWS, _LANES), lambda: (0, 0)),
            pl.BlockSpec((_IN_ROWS, _LANES), lambda: (0, 0)),
            pl.BlockSpec((1, 1), lambda: (0, 0), memory_space=pltpu.SMEM),
            pl.BlockSpec((1, 1), lambda: (0, 0), memory_space=pltpu.SMEM),
        ],
        out_specs=pl.BlockSpec((_ROWS, _LANES), lambda: (0, 0)),
        out_shape=jax.ShapeDtypeStruct((_ROWS, _LANES), jnp.int32),
        scratch_shapes=[
            pltpu.VMEM((_ROWS, _LANES), jnp.int32),
        ],
    )(att_p, idx_p, mn, mx)


def kernel(emb, edge_index, node_batch, W1, b1, W2, b2):
    row = edge_index[0]
    col = edge_index[1]
    R = jnp.take(emb, row, axis=0)
    C = jnp.take(emb, col, axis=0)
    att = _mlp_att(R, C, W1, b1, W2, b2)
    index = jnp.take(node_batch, row)

    mn = att.min().reshape(1, 1)
    mx = att.max().reshape(1, 1)
    att_p = jnp.pad(att, (0, _PAD_TO - N_EDGES)).reshape(_IN_ROWS, _LANES)
    idx_p = jnp.pad(index, (0, _PAD_TO - N_EDGES)).reshape(_IN_ROWS, _LANES)
    perm = _sort_perm(att_p, idx_p, mn, mx).reshape(_M)[:N_EDGES]

    deg = jnp.bincount(index, length=N_GRAPHS)
    k = jnp.ceil(RATIO * deg.astype(jnp.float32)).astype(jnp.int32)
    cum = jnp.cumsum(deg)
    start = jnp.concatenate([jnp.zeros((1,), dtype=deg.dtype), cum])
    pos = jnp.arange(N_EDGES)
    gpos = jnp.searchsorted(cum, pos, side='right')
    mask = (pos - jnp.take(start, gpos)) < jnp.take(k, gpos)
    sorted_att = jnp.take(att, perm)
    signed = jnp.where(mask, sorted_att, -sorted_att)

    ck = jnp.cumsum(k)
    k_total = ck[-1]
    cke = ck - k
    rest = deg.astype(jnp.int32) - k
    cr = jnp.cumsum(rest)
    cre = cr - rest
    gi = jnp.searchsorted(ck, pos, side='right')
    src_top = jnp.take(start, gi) + (pos - jnp.take(cke, gi))
    j2 = pos - k_total
    gj = jnp.searchsorted(cr, j2, side='right')
    src_bot = jnp.take(start, gj) + jnp.take(k, gj) + (j2 - jnp.take(cre, gj))
    order = jnp.where(pos < k_total, src_top, src_bot)
    return jnp.take(signed, order)


# speedup vs baseline: 1.1858x; 1.1831x over previous
import jax
import jax.numpy as jnp
from jax.experimental import pallas as pl
from jax.experimental.pallas import tpu as pltpu

N_EDGES = 320000
N_GRAPHS = 64
RATIO = 0.5
EPS = 1e-12

_BLK = 3200

_ROWS = 4096
_LANES = 128
_M = _ROWS * _LANES
_IN_ROWS = 2560
_PAD_TO = _IN_ROWS * _LANES
_IMAX = 2147483647
_CH = 64
_CHP = 256


def _mlp_body(r_ref, c_ref, w1a_ref, w1b_ref, b1_ref, w2_ref, b2_ref, att_ref):
    h = jnp.dot(r_ref[...], w1a_ref[...], preferred_element_type=jnp.float32)
    h = h + jnp.dot(c_ref[...], w1b_ref[...], preferred_element_type=jnp.float32)
    h = jax.nn.relu(h + b1_ref[...])
    att = jnp.sum(h * w2_ref[...], axis=1) + b2_ref[0, 0]
    att_ref[...] = att.reshape(1, 1, -1)


def _mlp_att(R, C, W1, b1, W2, b2):
    W1a = W1[:128]
    W1b = W1[128:]
    b1r = b1.reshape(1, -1)
    w2r = W2.reshape(1, -1)
    b2r = b2.reshape(1, 1)
    grid = N_EDGES // _BLK
    return pl.pallas_call(
        _mlp_body,
        grid=(grid,),
        in_specs=[
            pl.BlockSpec((_BLK, 128), lambda i: (i, 0)),
            pl.BlockSpec((_BLK, 128), lambda i: (i, 0)),
            pl.BlockSpec((128, 512), lambda i: (0, 0)),
            pl.BlockSpec((128, 512), lambda i: (0, 0)),
            pl.BlockSpec((1, 512), lambda i: (0, 0)),
            pl.BlockSpec((1, 512), lambda i: (0, 0)),
            pl.BlockSpec((1, 1), lambda i: (0, 0), memory_space=pltpu.SMEM),
        ],
        out_specs=pl.BlockSpec((1, 1, _BLK), lambda i: (i, 0, 0)),
        out_shape=jax.ShapeDtypeStruct((grid, 1, _BLK), jnp.float32),
    )(R, C, W1a, W1b, b1r, w2r, b2r).reshape(N_EDGES)


def _exch(K, V, kp, vp, right, desc):
    sgtp = (K > kp) | ((K == kp) & (V > vp))
    take = sgtp ^ right ^ desc
    return jnp.where(take, kp, K), jnp.where(take, vp, V)


def _sort_body(att_ref, idx_ref, mn_ref, mx_ref, perm_ref, k_ref):
    v_ref = perm_ref
    mn = mn_ref[0, 0]
    mx = mx_ref[0, 0]
    laneP = jax.lax.broadcasted_iota(jnp.int32, (1, _LANES), 1)
    lrowP = jax.lax.broadcasted_iota(jnp.int32, (_CHP, 1), 0)
    lrow = jax.lax.broadcasted_iota(jnp.int32, (_CH, 1), 0)

    def pro(q, c):
        base = pl.multiple_of(q * _CHP, _CHP)
        a = att_ref[pl.ds(base, _CHP), :]
        g = idx_ref[pl.ds(base, _CHP), :].astype(jnp.float32)
        norm = (a - mn) / (mx - mn + EPS) + g * (-1.0)
        u = jax.lax.bitcast_convert_type(norm, jnp.uint32)
        s = jnp.where(u < jnp.uint32(0x80000000), u ^ jnp.uint32(0x80000000), ~u)
        ki = jax.lax.bitcast_convert_type((~s) ^ jnp.uint32(0x80000000), jnp.int32)
        eid = (base + lrowP) * 128 + laneP
        valid = eid < N_EDGES
        k_ref[pl.ds(base, _CHP), :] = jnp.where(valid, ki, jnp.int32(_IMAX))
        v_ref[pl.ds(base, _CHP), :] = jnp.where(valid, eid, jnp.int32(_IMAX))
        return c

    jax.lax.fori_loop(0, _IN_ROWS // _CHP, pro, 0)

    padc = jnp.full((_CHP, _LANES), _IMAX, jnp.int32)

    def padf(q, c):
        base = pl.multiple_of(_IN_ROWS + q * _CHP, _CHP)
        k_ref[pl.ds(base, _CHP), :] = padc
        v_ref[pl.ds(base, _CHP), :] = padc
        return c

    jax.lax.fori_loop(0, (_ROWS - _IN_ROWS) // _CHP, padf, 0)

    lane = jax.lax.broadcasted_iota(jnp.int32, (1, _LANES), 1)

    def roll_exch(K, V, axis, r, right, desc):
        kp = jnp.where(right, jnp.roll(K, r, axis=axis), jnp.roll(K, -r, axis=axis))
        vp = jnp.where(right, jnp.roll(V, r, axis=axis), jnp.roll(V, -r, axis=axis))
        return _exch(K, V, kp, vp, right, desc)

    def phase_a(q, c):
        base = pl.multiple_of(q * _CH, _CH)
        K = k_ref[pl.ds(base, _CH), :]
        V = v_ref[pl.ds(base, _CH), :]
        for kk in range(1, 8):
            if kk < 7:
                desc = ((lane >> kk) & 1) != 0
            else:
                desc = (lrow & 1) != 0
            for j in range(min(kk - 1, 6), -1, -1):
                s2 = 1 << j
                right = (lane & s2) != 0
                K, V = roll_exch(K, V, 1, s2, right, desc)
        k_ref[pl.ds(base, _CH), :] = K
        v_ref[pl.ds(base, _CH), :] = V
        return c

    jax.lax.fori_loop(0, _ROWS // _CH, phase_a, 0)

    for kk in range(8, 20):
        for j in range(kk - 1, 12, -1):
            r = 1 << (j - 7)
            ch2 = min(r, 64)
            tpb = r // ch2

            def bigrow(q, c, r=r, ch2=ch2, tpb=tpb, kk=kk, j=j):
                bp = q // tpb
                t = q % tpb
                base = pl.multiple_of(bp * (2 * r) + t * ch2, 8)
                klo = k_ref[pl.ds(base, ch2), :]
                vlo = v_ref[pl.ds(base, ch2), :]
                khi = k_ref[pl.ds(base + r, ch2), :]
                vhi = v_ref[pl.ds(base + r, ch2), :]
                asc = ((bp >> (kk - j - 1)) & 1) == 0
                gt01 = (klo > khi) | ((klo == khi) & (vlo > vhi))
                lt01 = (khi > klo) | ((khi == klo) & (vhi > vlo))
                swap = (gt01 & asc) | (lt01 & jnp.logical_not(asc))
                k_ref[pl.ds(base, ch2), :] = jnp.where(swap, khi, klo)
                v_ref[pl.ds(base, ch2), :] = jnp.where(swap, vhi, vlo)
                k_ref[pl.ds(base + r, ch2), :] = jnp.where(swap, klo, khi)
                v_ref[pl.ds(base + r, ch2), :] = jnp.where(swap, vlo, vhi)
                return c

            jax.lax.fori_loop(0, (_ROWS // (2 * r)) * tpb, bigrow, 0)

        def tail(q, c, kk=kk):
            base = pl.multiple_of(q * _CH, _CH)
            K = k_ref[pl.ds(base, _CH), :]
            V = v_ref[pl.ds(base, _CH), :]
            kb = kk - 7
            if (1 << kb) < _CH:
                desc = (((base + lrow) >> kb) & 1) != 0
            else:
                desc = (((base >> kb) & 1) != 0)
            for j in range(min(kk - 1, 12), -1, -1):
                if j >= 7:
                    r = 1 << (j - 7)
                    right = (lrow & r) != 0
                    K, V = roll_exch(K, V, 0, r, right, desc)
                else:
                    s2 = 1 << j
                    right = (lane & s2) != 0
                    K, V = roll_exch(K, V, 1, s2, right, desc)
            k_ref[pl.ds(base, _CH), :] = K
            v_ref[pl.ds(base, _CH), :] = V
            return c

        jax.lax.fori_loop(0, _ROWS // _CH, tail, 0)


def _sort_perm(att_p, idx_p, mn, mx):
    return pl.pallas_call(
        _sort_body,
        in_specs=[
            pl.BlockSpec((_IN_ROWS, _LANES), lambda: (0, 0)),
            pl.BlockSpec((_IN_ROWS, _LANES), lambda: (0, 0)),
            pl.BlockSpec((1, 1), lambda: (0, 0), memory_space=pltpu.SMEM),
            pl.BlockSpec((1, 1), lambda: (0, 0), memory_space=pltpu.SMEM),
        ],
        out_specs=pl.BlockSpec((_ROWS, _LANES), lambda: (0, 0)),
        out_shape=jax.ShapeDtypeStruct((_ROWS, _LANES), jnp.int32),
        scratch_shapes=[
            pltpu.VMEM((_ROWS, _LANES), jnp.int32),
        ],
    )(att_p, idx_p, mn, mx)


def kernel(emb, edge_index, node_batch, W1, b1, W2, b2):
    row = edge_index[0]
    col = edge_index[1]
    R = jnp.take(emb, row, axis=0)
    C = jnp.take(emb, col, axis=0)
    att = _mlp_att(R, C, W1, b1, W2, b2)
    index = jnp.take(node_batch, row)

    mn = att.min().reshape(1, 1)
    mx = att.max().reshape(1, 1)
    att_p = jnp.pad(att, (0, _PAD_TO - N_EDGES)).reshape(_IN_ROWS, _LANES)
    idx_p = jnp.pad(index, (0, _PAD_TO - N_EDGES)).reshape(_IN_ROWS, _LANES)
    perm = _sort_perm(att_p, idx_p, mn, mx).reshape(_M)[:N_EDGES]

    deg = jnp.bincount(index, length=N_GRAPHS)
    k = jnp.ceil(RATIO * deg.astype(jnp.float32)).astype(jnp.int32)
    cum = jnp.cumsum(deg)
    start = jnp.concatenate([jnp.zeros((1,), dtype=deg.dtype), cum])
    pos = jnp.arange(N_EDGES)
    gpos = jnp.sum(pos[:, None] >= cum[None, :], axis=1, dtype=jnp.int32)
    mask = (pos - jnp.take(start, gpos)) < jnp.take(k, gpos)
    sorted_att = jnp.take(att, perm)
    signed = jnp.where(mask, sorted_att, -sorted_att)

    ck = jnp.cumsum(k)
    k_total = ck[-1]
    cke = ck - k
    rest = deg.astype(jnp.int32) - k
    cr = jnp.cumsum(rest)
    cre = cr - rest
    gi = jnp.sum(pos[:, None] >= ck[None, :], axis=1, dtype=jnp.int32)
    src_top = jnp.take(start, gi) + (pos - jnp.take(cke, gi))
    j2 = pos - k_total
    gj = jnp.sum(j2[:, None] >= cr[None, :], axis=1, dtype=jnp.int32)
    src_bot = jnp.take(start, gj) + jnp.take(k, gj) + (j2 - jnp.take(cre, gj))
    order = jnp.where(pos < k_total, src_top, src_bot)
    return jnp.take(signed, order)


# speedup vs baseline: 2.8417x; 2.3965x over previous
import jax
import jax.numpy as jnp
from jax.experimental import pallas as pl
from jax.experimental.pallas import tpu as pltpu

N_EDGES = 320000
N_GRAPHS = 64
RATIO = 0.5
EPS = 1e-12

_BLK = 3200

_ROWS = 4096
_LANES = 128
_M = _ROWS * _LANES
_IN_ROWS = 2560
_PAD_TO = _IN_ROWS * _LANES
_IMAX = 2147483647
_CH = 64
_CHP = 256


def _mlp_body(r_ref, c_ref, w1a_ref, w1b_ref, b1_ref, w2_ref, b2_ref, att_ref):
    h = jnp.dot(r_ref[...], w1a_ref[...], preferred_element_type=jnp.float32)
    h = h + jnp.dot(c_ref[...], w1b_ref[...], preferred_element_type=jnp.float32)
    h = jax.nn.relu(h + b1_ref[...])
    att = jnp.sum(h * w2_ref[...], axis=1) + b2_ref[0, 0]
    att_ref[...] = att.reshape(1, 1, -1)


def _mlp_att(R, C, W1, b1, W2, b2):
    W1a = W1[:128]
    W1b = W1[128:]
    b1r = b1.reshape(1, -1)
    w2r = W2.reshape(1, -1)
    b2r = b2.reshape(1, 1)
    grid = N_EDGES // _BLK
    return pl.pallas_call(
        _mlp_body,
        grid=(grid,),
        in_specs=[
            pl.BlockSpec((_BLK, 128), lambda i: (i, 0)),
            pl.BlockSpec((_BLK, 128), lambda i: (i, 0)),
            pl.BlockSpec((128, 512), lambda i: (0, 0)),
            pl.BlockSpec((128, 512), lambda i: (0, 0)),
            pl.BlockSpec((1, 512), lambda i: (0, 0)),
            pl.BlockSpec((1, 512), lambda i: (0, 0)),
            pl.BlockSpec((1, 1), lambda i: (0, 0), memory_space=pltpu.SMEM),
        ],
        out_specs=pl.BlockSpec((1, 1, _BLK), lambda i: (i, 0, 0)),
        out_shape=jax.ShapeDtypeStruct((grid, 1, _BLK), jnp.float32),
    )(R, C, W1a, W1b, b1r, w2r, b2r).reshape(N_EDGES)


def _exch(K, V, kp, vp, right, desc):
    sgtp = (K > kp) | ((K == kp) & (V > vp))
    take = sgtp ^ right ^ desc
    return jnp.where(take, kp, K), jnp.where(take, vp, V)


def _sort_body(att_ref, idx_ref, mn_ref, mx_ref, perm_ref, k_ref):
    v_ref = perm_ref
    mn = mn_ref[0, 0]
    mx = mx_ref[0, 0]
    laneP = jax.lax.broadcasted_iota(jnp.int32, (1, _LANES), 1)
    lrowP = jax.lax.broadcasted_iota(jnp.int32, (_CHP, 1), 0)
    lrow = jax.lax.broadcasted_iota(jnp.int32, (_CH, 1), 0)

    def pro(q, c):
        base = pl.multiple_of(q * _CHP, _CHP)
        a = att_ref[pl.ds(base, _CHP), :]
        g = idx_ref[pl.ds(base, _CHP), :].astype(jnp.float32)
        norm = (a - mn) / (mx - mn + EPS) + g * (-1.0)
        u = jax.lax.bitcast_convert_type(norm, jnp.uint32)
        s = jnp.where(u < jnp.uint32(0x80000000), u ^ jnp.uint32(0x80000000), ~u)
        ki = jax.lax.bitcast_convert_type((~s) ^ jnp.uint32(0x80000000), jnp.int32)
        eid = (base + lrowP) * 128 + laneP
        valid = eid < N_EDGES
        k_ref[pl.ds(base, _CHP), :] = jnp.where(valid, ki, jnp.int32(_IMAX))
        v_ref[pl.ds(base, _CHP), :] = jnp.where(valid, eid, jnp.int32(_IMAX))
        return c

    jax.lax.fori_loop(0, _IN_ROWS // _CHP, pro, 0)

    padc = jnp.full((_CHP, _LANES), _IMAX, jnp.int32)

    def padf(q, c):
        base = pl.multiple_of(_IN_ROWS + q * _CHP, _CHP)
        k_ref[pl.ds(base, _CHP), :] = padc
        v_ref[pl.ds(base, _CHP), :] = padc
        return c

    jax.lax.fori_loop(0, (_ROWS - _IN_ROWS) // _CHP, padf, 0)

    lane = jax.lax.broadcasted_iota(jnp.int32, (1, _LANES), 1)

    def roll_exch(K, V, axis, r, right, desc):
        kp = jnp.where(right, jnp.roll(K, r, axis=axis), jnp.roll(K, -r, axis=axis))
        vp = jnp.where(right, jnp.roll(V, r, axis=axis), jnp.roll(V, -r, axis=axis))
        return _exch(K, V, kp, vp, right, desc)

    def phase_a(q, c):
        base = pl.multiple_of(q * _CH, _CH)
        K = k_ref[pl.ds(base, _CH), :]
        V = v_ref[pl.ds(base, _CH), :]
        for kk in range(1, 8):
            if kk < 7:
                desc = ((lane >> kk) & 1) != 0
            else:
                desc = (lrow & 1) != 0
            for j in range(min(kk - 1, 6), -1, -1):
                s2 = 1 << j
                right = (lane & s2) != 0
                K, V = roll_exch(K, V, 1, s2, right, desc)
        k_ref[pl.ds(base, _CH), :] = K
        v_ref[pl.ds(base, _CH), :] = V
        return c

    jax.lax.fori_loop(0, _ROWS // _CH, phase_a, 0)

    for kk in range(8, 20):
        for j in range(kk - 1, 12, -1):
            r = 1 << (j - 7)
            ch2 = min(r, 64)
            tpb = r // ch2

            def bigrow(q, c, r=r, ch2=ch2, tpb=tpb, kk=kk, j=j):
                bp = q // tpb
                t = q % tpb
                base = pl.multiple_of(bp * (2 * r) + t * ch2, 8)
                klo = k_ref[pl.ds(base, ch2), :]
                vlo = v_ref[pl.ds(base, ch2), :]
                khi = k_ref[pl.ds(base + r, ch2), :]
                vhi = v_ref[pl.ds(base + r, ch2), :]
                asc = ((bp >> (kk - j - 1)) & 1) == 0
                gt01 = (klo > khi) | ((klo == khi) & (vlo > vhi))
                lt01 = (khi > klo) | ((khi == klo) & (vhi > vlo))
                swap = (gt01 & asc) | (lt01 & jnp.logical_not(asc))
                k_ref[pl.ds(base, ch2), :] = jnp.where(swap, khi, klo)
                v_ref[pl.ds(base, ch2), :] = jnp.where(swap, vhi, vlo)
                k_ref[pl.ds(base + r, ch2), :] = jnp.where(swap, klo, khi)
                v_ref[pl.ds(base + r, ch2), :] = jnp.where(swap, vlo, vhi)
                return c

            jax.lax.fori_loop(0, (_ROWS // (2 * r)) * tpb, bigrow, 0)

        def tail(q, c, kk=kk):
            base = pl.multiple_of(q * _CH, _CH)
            K = k_ref[pl.ds(base, _CH), :]
            V = v_ref[pl.ds(base, _CH), :]
            kb = kk - 7
            if (1 << kb) < _CH:
                desc = (((base + lrow) >> kb) & 1) != 0
            else:
                desc = (((base >> kb) & 1) != 0)
            for j in range(min(kk - 1, 12), -1, -1):
                if j >= 7:
                    r = 1 << (j - 7)
                    right = (lrow & r) != 0
                    K, V = roll_exch(K, V, 0, r, right, desc)
                else:
                    s2 = 1 << j
                    right = (lane & s2) != 0
                    K, V = roll_exch(K, V, 1, s2, right, desc)
            k_ref[pl.ds(base, _CH), :] = K
            v_ref[pl.ds(base, _CH), :] = V
            return c

        jax.lax.fori_loop(0, _ROWS // _CH, tail, 0)


def _sort_perm(att_p, idx_p, mn, mx):
    return pl.pallas_call(
        _sort_body,
        in_specs=[
            pl.BlockSpec((_IN_ROWS, _LANES), lambda: (0, 0)),
            pl.BlockSpec((_IN_ROWS, _LANES), lambda: (0, 0)),
            pl.BlockSpec((1, 1), lambda: (0, 0), memory_space=pltpu.SMEM),
            pl.BlockSpec((1, 1), lambda: (0, 0), memory_space=pltpu.SMEM),
        ],
        out_specs=pl.BlockSpec((_ROWS, _LANES), lambda: (0, 0)),
        out_shape=jax.ShapeDtypeStruct((_ROWS, _LANES), jnp.int32),
        scratch_shapes=[
            pltpu.VMEM((_ROWS, _LANES), jnp.int32),
        ],
    )(att_p, idx_p, mn, mx)


def kernel(emb, edge_index, node_batch, W1, b1, W2, b2):
    row = edge_index[0]
    col = edge_index[1]
    R = jnp.take(emb, row, axis=0)
    C = jnp.take(emb, col, axis=0)
    att = _mlp_att(R, C, W1, b1, W2, b2)
    index = jnp.take(node_batch, row)

    mn = att.min().reshape(1, 1)
    mx = att.max().reshape(1, 1)
    att_p = jnp.pad(att, (0, _PAD_TO - N_EDGES)).reshape(_IN_ROWS, _LANES)
    idx_p = jnp.pad(index, (0, _PAD_TO - N_EDGES)).reshape(_IN_ROWS, _LANES)
    perm = _sort_perm(att_p, idx_p, mn, mx).reshape(_M)[:N_EDGES]

    return jnp.take(att, perm)


# speedup vs baseline: 6.7338x; 2.3696x over previous
import jax
import jax.numpy as jnp
from jax.experimental import pallas as pl
from jax.experimental.pallas import tpu as pltpu

N_EDGES = 320000
N_GRAPHS = 64
RATIO = 0.5
EPS = 1e-12

_BLK = 3200

_ROWS = 4096
_LANES = 128
_M = _ROWS * _LANES
_IN_ROWS = 2560
_PAD_TO = _IN_ROWS * _LANES
_IMAX = 2147483647
_CH = 64
_CHP = 256


def _mlp_body(r_ref, c_ref, w1a_ref, w1b_ref, b1_ref, w2_ref, b2_ref, att_ref):
    h = jnp.dot(r_ref[...], w1a_ref[...], preferred_element_type=jnp.float32)
    h = h + jnp.dot(c_ref[...], w1b_ref[...], preferred_element_type=jnp.float32)
    h = jax.nn.relu(h + b1_ref[...])
    att = jnp.sum(h * w2_ref[...], axis=1) + b2_ref[0, 0]
    att_ref[...] = att.reshape(1, 1, -1)


def _mlp_att(R, C, W1, b1, W2, b2):
    W1a = W1[:128]
    W1b = W1[128:]
    b1r = b1.reshape(1, -1)
    w2r = W2.reshape(1, -1)
    b2r = b2.reshape(1, 1)
    grid = N_EDGES // _BLK
    return pl.pallas_call(
        _mlp_body,
        grid=(grid,),
        in_specs=[
            pl.BlockSpec((_BLK, 128), lambda i: (i, 0)),
            pl.BlockSpec((_BLK, 128), lambda i: (i, 0)),
            pl.BlockSpec((128, 512), lambda i: (0, 0)),
            pl.BlockSpec((128, 512), lambda i: (0, 0)),
            pl.BlockSpec((1, 512), lambda i: (0, 0)),
            pl.BlockSpec((1, 512), lambda i: (0, 0)),
            pl.BlockSpec((1, 1), lambda i: (0, 0), memory_space=pltpu.SMEM),
        ],
        out_specs=pl.BlockSpec((1, 1, _BLK), lambda i: (i, 0, 0)),
        out_shape=jax.ShapeDtypeStruct((grid, 1, _BLK), jnp.float32),
    )(R, C, W1a, W1b, b1r, w2r, b2r).reshape(N_EDGES)


def _exch(K, V, kp, vp, right, desc):
    sgtp = (K > kp) | ((K == kp) & (V > vp))
    take = sgtp ^ right ^ desc
    return jnp.where(take, kp, K), jnp.where(take, vp, V)


def _sort_body(att_ref, idx_ref, mn_ref, mx_ref, perm_ref, k_ref):
    v_ref = perm_ref
    mn = mn_ref[0, 0]
    mx = mx_ref[0, 0]
    laneP = jax.lax.broadcasted_iota(jnp.int32, (1, _LANES), 1)
    lrowP = jax.lax.broadcasted_iota(jnp.int32, (_CHP, 1), 0)
    lrow = jax.lax.broadcasted_iota(jnp.int32, (_CH, 1), 0)

    def pro(q, c):
        base = pl.multiple_of(q * _CHP, _CHP)
        a = att_ref[pl.ds(base, _CHP), :]
        g = idx_ref[pl.ds(base, _CHP), :].astype(jnp.float32)
        norm = (a - mn) / (mx - mn + EPS) + g * (-1.0)
        u = jax.lax.bitcast_convert_type(norm, jnp.uint32)
        s = jnp.where(u < jnp.uint32(0x80000000), u ^ jnp.uint32(0x80000000), ~u)
        ki = jax.lax.bitcast_convert_type((~s) ^ jnp.uint32(0x80000000), jnp.int32)
        eid = (base + lrowP) * 128 + laneP
        valid = eid < N_EDGES
        k_ref[pl.ds(base, _CHP), :] = jnp.where(valid, ki, jnp.int32(_IMAX))
        v_ref[pl.ds(base, _CHP), :] = jnp.where(valid, eid, jnp.int32(_IMAX))
        return c

    jax.lax.fori_loop(0, _IN_ROWS // _CHP, pro, 0)

    padc = jnp.full((_CHP, _LANES), _IMAX, jnp.int32)

    def padf(q, c):
        base = pl.multiple_of(_IN_ROWS + q * _CHP, _CHP)
        k_ref[pl.ds(base, _CHP), :] = padc
        v_ref[pl.ds(base, _CHP), :] = padc
        return c

    jax.lax.fori_loop(0, (_ROWS - _IN_ROWS) // _CHP, padf, 0)

    lane = jax.lax.broadcasted_iota(jnp.int32, (1, _LANES), 1)

    def roll_exch(K, V, axis, r, right, desc):
        kp = jnp.where(right, jnp.roll(K, r, axis=axis), jnp.roll(K, -r, axis=axis))
        vp = jnp.where(right, jnp.roll(V, r, axis=axis), jnp.roll(V, -r, axis=axis))
        return _exch(K, V, kp, vp, right, desc)

    def phase_a(q, c):
        base = pl.multiple_of(q * _CH, _CH)
        K = k_ref[pl.ds(base, _CH), :]
        V = v_ref[pl.ds(base, _CH), :]
        for kk in range(1, 8):
            if kk < 7:
                desc = ((lane >> kk) & 1) != 0
            else:
                desc = (lrow & 1) != 0
            for j in range(min(kk - 1, 6), -1, -1):
                s2 = 1 << j
                right = (lane & s2) != 0
                K, V = roll_exch(K, V, 1, s2, right, desc)
        k_ref[pl.ds(base, _CH), :] = K
        v_ref[pl.ds(base, _CH), :] = V
        return c

    jax.lax.fori_loop(0, _ROWS // _CH, phase_a, 0)

    for kk in range(8, 20):
        for j in range(kk - 1, 12, -1):
            r = 1 << (j - 7)
            ch2 = min(r, 64)
            tpb = r // ch2

            def bigrow(q, c, r=r, ch2=ch2, tpb=tpb, kk=kk, j=j):
                bp = q // tpb
                t = q % tpb
                base = pl.multiple_of(bp * (2 * r) + t * ch2, 8)
                klo = k_ref[pl.ds(base, ch2), :]
                vlo = v_ref[pl.ds(base, ch2), :]
                khi = k_ref[pl.ds(base + r, ch2), :]
                vhi = v_ref[pl.ds(base + r, ch2), :]
                asc = ((bp >> (kk - j - 1)) & 1) == 0
                gt01 = (klo > khi) | ((klo == khi) & (vlo > vhi))
                lt01 = (khi > klo) | ((khi == klo) & (vhi > vlo))
                swap = (gt01 & asc) | (lt01 & jnp.logical_not(asc))
                k_ref[pl.ds(base, ch2), :] = jnp.where(swap, khi, klo)
                v_ref[pl.ds(base, ch2), :] = jnp.where(swap, vhi, vlo)
                k_ref[pl.ds(base + r, ch2), :] = jnp.where(swap, klo, khi)
                v_ref[pl.ds(base + r, ch2), :] = jnp.where(swap, vlo, vhi)
                return c

            jax.lax.fori_loop(0, (_ROWS // (2 * r)) * tpb, bigrow, 0)

        def tail(q, c, kk=kk):
            base = pl.multiple_of(q * _CH, _CH)
            K = k_ref[pl.ds(base, _CH), :]
            V = v_ref[pl.ds(base, _CH), :]
            kb = kk - 7
            if (1 << kb) < _CH:
                desc = (((base + lrow) >> kb) & 1) != 0
            else:
                desc = (((base >> kb) & 1) != 0)
            for j in range(min(kk - 1, 12), -1, -1):
                if j >= 7:
                    r = 1 << (j - 7)
                    right = (lrow & r) != 0
                    K, V = roll_exch(K, V, 0, r, right, desc)
                else:
                    s2 = 1 << j
                    right = (lane & s2) != 0
                    K, V = roll_exch(K, V, 1, s2, right, desc)
            k_ref[pl.ds(base, _CH), :] = K
            v_ref[pl.ds(base, _CH), :] = V
            return c

        jax.lax.fori_loop(0, _ROWS // _CH, tail, 0)


def _sort_perm(att_p, idx_p, mn, mx):
    return pl.pallas_call(
        _sort_body,
        in_specs=[
            pl.BlockSpec((_IN_ROWS, _LANES), lambda: (0, 0)),
            pl.BlockSpec((_IN_ROWS, _LANES), lambda: (0, 0)),
            pl.BlockSpec((1, 1), lambda: (0, 0), memory_space=pltpu.SMEM),
            pl.BlockSpec((1, 1), lambda: (0, 0), memory_space=pltpu.SMEM),
        ],
        out_specs=pl.BlockSpec((_ROWS, _LANES), lambda: (0, 0)),
        out_shape=jax.ShapeDtypeStruct((_ROWS, _LANES), jnp.int32),
        scratch_shapes=[
            pltpu.VMEM((_ROWS, _LANES), jnp.int32),
        ],
    )(att_p, idx_p, mn, mx)


def kernel(emb, edge_index, node_batch, W1, b1, W2, b2):
    row = edge_index[0]
    col = edge_index[1]
    R = jnp.take(emb, row, axis=0)
    C = jnp.take(emb, col, axis=0)
    att = _mlp_att(R, C, W1, b1, W2, b2)
    index = jnp.take(node_batch, row)

    return att
